# baseline (speedup 1.0000x reference)
import functools

import jax
import jax.numpy as jnp
from jax import lax
from jax.experimental import pallas as pl
from jax.experimental.pallas import tpu as pltpu
from jax.experimental.pallas import tpu_sc as plsc

N = 10000
E = 160000
F = 128

_LOG2 = 0.6931471805599453


def _ssp(z):
    return jnp.maximum(z, 0.0) + jnp.log1p(jnp.exp(-jnp.abs(z))) - _LOG2


_BE = 1600


def _filter_body(dijk_ref, w1_ref, b1_ref, w2_ref, b2_ref, out_ref):
    h = jnp.dot(dijk_ref[...], w1_ref[...], preferred_element_type=jnp.float32)
    h = _ssp(h + b1_ref[...])
    w = jnp.dot(h, w2_ref[...], preferred_element_type=jnp.float32)
    out_ref[...] = _ssp(w + b2_ref[...])


def _filter(dijk, W1, b1, W2, b2):
    return pl.pallas_call(
        _filter_body,
        grid=(E // _BE,),
        in_specs=[
            pl.BlockSpec((_BE, F), lambda i: (i, 0)),
            pl.BlockSpec((F, F), lambda i: (0, 0)),
            pl.BlockSpec((1, F), lambda i: (0, 0)),
            pl.BlockSpec((F, F), lambda i: (0, 0)),
            pl.BlockSpec((1, F), lambda i: (0, 0)),
        ],
        out_specs=pl.BlockSpec((_BE, F), lambda i: (i, 0)),
        out_shape=jax.ShapeDtypeStruct((E, F), jnp.float32),
    )(dijk, W1, b1.reshape(1, F), W2, b2.reshape(1, F))


_BN = 1000


def _in2fac_body(x_ref, win_ref, f_ref):
    f_ref[...] = jnp.dot(x_ref[...], win_ref[...],
                         preferred_element_type=jnp.float32)


def _in2fac(x, Win):
    return pl.pallas_call(
        _in2fac_body,
        grid=(N // _BN,),
        in_specs=[
            pl.BlockSpec((_BN, F), lambda i: (i, 0)),
            pl.BlockSpec((F, F), lambda i: (0, 0)),
        ],
        out_specs=pl.BlockSpec((_BN, F), lambda i: (i, 0)),
        out_shape=jax.ShapeDtypeStruct((N, F), jnp.float32),
    )(x, Win)


_NC = 2
_NS = 16
_NW = _NC * _NS
_CHUNK = 40
_EPT = E // _NW
_STEPS = _EPT // _CHUNK
_NBUF = 3
_NPAD = 10240
_ROWS_PER_TILE = _NPAD // _NS


def _sc_conv_body(f_hbm, w_hbm, idx_hbm, seg_hbm, zeros_hbm, out_hbm,
                  idx_v, seg_v, rows_v, w_v, conv_sh,
                  sem_i, sem_s, sem_g, sem_w, sem_sc):
    cid = lax.axis_index("c")
    sid = lax.axis_index("s")
    wid = cid * _NS + sid
    ebase = wid * _EPT

    pltpu.sync_copy(zeros_hbm.at[pl.ds(sid * _ROWS_PER_TILE, _ROWS_PER_TILE)],
                    conv_sh.at[pl.ds(sid * _ROWS_PER_TILE, _ROWS_PER_TILE)])
    plsc.subcore_barrier()

    def issue_idx(k, slot):
        pltpu.async_copy(idx_hbm.at[pl.ds(ebase + k * _CHUNK, _CHUNK)],
                         idx_v.at[slot], sem_i.at[slot])
        pltpu.async_copy(seg_hbm.at[pl.ds(ebase + k * _CHUNK, _CHUNK)],
                         seg_v.at[slot], sem_s.at[slot])

    def wait_idx(slot):
        pltpu.make_async_copy(idx_hbm.at[pl.ds(0, _CHUNK)], idx_v.at[slot],
                              sem_i.at[slot]).wait()
        pltpu.make_async_copy(idx_hbm.at[pl.ds(0, _CHUNK)], seg_v.at[slot],
                              sem_s.at[slot]).wait()

    def issue_loads(k, slot):
        pltpu.async_copy(f_hbm.at[idx_v.at[slot]], rows_v.at[slot],
                         sem_g.at[slot])
        pltpu.async_copy(w_hbm.at[pl.ds(ebase + k * _CHUNK, _CHUNK)],
                         w_v.at[slot], sem_w.at[slot])

    def wait_loads(slot):
        pltpu.make_async_copy(w_hbm.at[pl.ds(0, _CHUNK)], rows_v.at[slot],
                              sem_g.at[slot]).wait()
        pltpu.make_async_copy(w_hbm.at[pl.ds(0, _CHUNK)], w_v.at[slot],
                              sem_w.at[slot]).wait()

    def wait_scatter(slot):
        pltpu.make_async_copy(w_hbm.at[pl.ds(0, _CHUNK)], rows_v.at[slot],
                              sem_sc.at[slot]).wait()

    issue_idx(0, 0)
    issue_idx(1, 1)
    issue_idx(2, 2)
    wait_idx(0)
    issue_loads(0, 0)

    def step(k, _):
        slot = lax.rem(k, _NBUF)
        nslot = lax.rem(k + 1, _NBUF)

        @pl.when(k + 1 < _STEPS)
        def _():
            wait_idx(nslot)
            issue_loads(k + 1, nslot)

        wait_loads(slot)

        def mul_row(e, _):
            for c in range(F // 16):
                sl = pl.ds(c * 16, 16)
                rows_v[slot, e, sl] = rows_v[slot, e, sl] * w_v[slot, e, sl]
            return 0

        lax.fori_loop(0, _CHUNK, mul_row, 0)
        pltpu.async_copy(rows_v.at[slot], conv_sh.at[seg_v.at[slot]],
                         sem_sc.at[slot], add=True)

        @pl.when(k >= 1)
        def _():
            wait_scatter(lax.rem(k - 1, _NBUF))

            @pl.when(k + 2 < _STEPS)
            def _():
                issue_idx(k + 2, lax.rem(k + 2, _NBUF))

        return 0

    lax.fori_loop(0, _STEPS, step, 0)
    wait_scatter((_STEPS - 1) % _NBUF)
    plsc.subcore_barrier()

    off = sid * _ROWS_PER_TILE
    pltpu.sync_copy(conv_sh.at[pl.ds(off, _ROWS_PER_TILE)],
                    out_hbm.at[cid, pl.ds(off, _ROWS_PER_TILE)])


def _sc_conv(f, w, idx_j, seg_i, zeros):
    mesh = plsc.VectorSubcoreMesh(core_axis_name="c", subcore_axis_name="s")
    kern = functools.partial(
        pl.kernel,
        out_type=jax.ShapeDtypeStruct((_NC, _NPAD, F), jnp.float32),
        mesh=mesh,
        scratch_types=[
            pltpu.VMEM((_NBUF, _CHUNK), jnp.int32),
            pltpu.VMEM((_NBUF, _CHUNK), jnp.int32),
            pltpu.VMEM((_NBUF, _CHUNK, F), jnp.float32),
            pltpu.VMEM((_NBUF, _CHUNK, F), jnp.float32),
            pltpu.VMEM_SHARED((_NPAD, F), jnp.float32),
            pltpu.SemaphoreType.DMA((_NBUF,)),
            pltpu.SemaphoreType.DMA((_NBUF,)),
            pltpu.SemaphoreType.DMA((_NBUF,)),
            pltpu.SemaphoreType.DMA((_NBUF,)),
            pltpu.SemaphoreType.DMA((_NBUF,)),
        ],
    )(_sc_conv_body)
    return kern(f, w, idx_j, seg_i, zeros)


def _epilogue_body(p0_ref, p1_ref, x_ref, wout_ref, bout_ref, wd_ref, bd_ref,
                   y_ref, v_ref):
    conv = p0_ref[0] + p1_ref[0]
    c = _ssp(jnp.dot(conv, wout_ref[...], preferred_element_type=jnp.float32)
             + bout_ref[...])
    v = jnp.dot(c, wd_ref[...], preferred_element_type=jnp.float32) + bd_ref[...]
    v_ref[...] = v
    y_ref[...] = x_ref[...] + v


def _epilogue(parts, x, Wout, bout, Wd, bd):
    nb = N // _BN
    return pl.pallas_call(
        _epilogue_body,
        grid=(nb,),
        in_specs=[
            pl.BlockSpec((1, _BN, F), lambda i: (0, i, 0)),
            pl.BlockSpec((1, _BN, F), lambda i: (1, i, 0)),
            pl.BlockSpec((_BN, F), lambda i: (i, 0)),
            pl.BlockSpec((F, F), lambda i: (0, 0)),
            pl.BlockSpec((1, F), lambda i: (0, 0)),
            pl.BlockSpec((F, F), lambda i: (0, 0)),
            pl.BlockSpec((1, F), lambda i: (0, 0)),
        ],
        out_specs=[
            pl.BlockSpec((_BN, F), lambda i: (i, 0)),
            pl.BlockSpec((_BN, F), lambda i: (i, 0)),
        ],
        out_shape=[
            jax.ShapeDtypeStruct((N, F), jnp.float32),
            jax.ShapeDtypeStruct((N, F), jnp.float32),
        ],
    )(parts, parts, x, Wout, bout.reshape(1, F), Wd, bd.reshape(1, F))


def kernel(x, dijk, idx_j, seg_i, seg_j, seg_i_sum,
           W1, b1, W2, b2, Win, Wout, bout, Wd, bd):
    w = _filter(dijk, W1, b1, W2, b2)
    f = _in2fac(x, Win)
    zeros = jnp.zeros((_NPAD, F), jnp.float32)
    parts = _sc_conv(f, w, idx_j.astype(jnp.int32), seg_i.astype(jnp.int32),
                     zeros)
    y, v = _epilogue(parts, x, Wout, bout, Wd, bd)
    return (y, v)


# speedup vs baseline: 1.5435x; 1.5435x over previous
import functools

import jax
import jax.numpy as jnp
from jax import lax
from jax.experimental import pallas as pl
from jax.experimental.pallas import tpu as pltpu
from jax.experimental.pallas import tpu_sc as plsc

N = 10000
E = 160000
F = 128

_LOG2 = 0.6931471805599453


def _ssp(z):
    return jnp.maximum(z, 0.0) + jnp.log1p(jnp.exp(-jnp.abs(z))) - _LOG2


_BE = 1600


def _filter_body(dijk_ref, w1_ref, b1_ref, w2_ref, b2_ref, out_ref):
    h = jnp.dot(dijk_ref[...], w1_ref[...], preferred_element_type=jnp.float32)
    h = _ssp(h + b1_ref[...])
    w = jnp.dot(h, w2_ref[...], preferred_element_type=jnp.float32)
    out_ref[...] = _ssp(w + b2_ref[...])


def _filter(dijk, W1, b1, W2, b2):
    return pl.pallas_call(
        _filter_body,
        grid=(E // _BE,),
        in_specs=[
            pl.BlockSpec((_BE, F), lambda i: (i, 0)),
            pl.BlockSpec((F, F), lambda i: (0, 0)),
            pl.BlockSpec((1, F), lambda i: (0, 0)),
            pl.BlockSpec((F, F), lambda i: (0, 0)),
            pl.BlockSpec((1, F), lambda i: (0, 0)),
        ],
        out_specs=pl.BlockSpec((_BE, F), lambda i: (i, 0)),
        out_shape=jax.ShapeDtypeStruct((E, F), jnp.float32),
    )(dijk, W1, b1.reshape(1, F), W2, b2.reshape(1, F))


_BN = 1000


def _in2fac_body(x_ref, win_ref, f_ref):
    f_ref[...] = jnp.dot(x_ref[...], win_ref[...],
                         preferred_element_type=jnp.float32)


def _in2fac(x, Win):
    return pl.pallas_call(
        _in2fac_body,
        grid=(N // _BN,),
        in_specs=[
            pl.BlockSpec((_BN, F), lambda i: (i, 0)),
            pl.BlockSpec((F, F), lambda i: (0, 0)),
        ],
        out_specs=pl.BlockSpec((_BN, F), lambda i: (i, 0)),
        out_shape=jax.ShapeDtypeStruct((N, F), jnp.float32),
    )(x, Win)


_NC = 2
_NS = 16
_NW = _NC * _NS
_CHUNK = 40
_EPT = E // _NW
_STEPS = _EPT // _CHUNK
_NBUF = 3
_NPAD = 10240
_ROWS_PER_TILE = _NPAD // _NS


def _sc_conv_body(f_hbm, w_hbm, idx_hbm, seg_hbm, zeros_hbm, out_hbm,
                  idx_v, seg_v, rows_v, w_v, conv_sh,
                  sem_i, sem_s, sem_g, sem_w, sem_sc):
    cid = lax.axis_index("c")
    sid = lax.axis_index("s")
    wid = cid * _NS + sid
    ebase = wid * _EPT

    pltpu.sync_copy(zeros_hbm.at[pl.ds(sid * _ROWS_PER_TILE, _ROWS_PER_TILE)],
                    conv_sh.at[pl.ds(sid * _ROWS_PER_TILE, _ROWS_PER_TILE)])
    plsc.subcore_barrier()

    def issue_idx(k, slot):
        pltpu.async_copy(idx_hbm.at[pl.ds(ebase + k * _CHUNK, _CHUNK)],
                         idx_v.at[slot], sem_i.at[slot])
        pltpu.async_copy(seg_hbm.at[pl.ds(ebase + k * _CHUNK, _CHUNK)],
                         seg_v.at[slot], sem_s.at[slot])

    def wait_idx(slot):
        pltpu.make_async_copy(idx_hbm.at[pl.ds(0, _CHUNK)], idx_v.at[slot],
                              sem_i.at[slot]).wait()
        pltpu.make_async_copy(idx_hbm.at[pl.ds(0, _CHUNK)], seg_v.at[slot],
                              sem_s.at[slot]).wait()

    def issue_loads(k, slot):
        pltpu.async_copy(f_hbm.at[idx_v.at[slot]], rows_v.at[slot],
                         sem_g.at[slot])
        pltpu.async_copy(w_hbm.at[pl.ds(ebase + k * _CHUNK, _CHUNK)],
                         w_v.at[slot], sem_w.at[slot])

    def wait_loads(slot):
        pltpu.make_async_copy(w_hbm.at[pl.ds(0, _CHUNK)], rows_v.at[slot],
                              sem_g.at[slot]).wait()
        pltpu.make_async_copy(w_hbm.at[pl.ds(0, _CHUNK)], w_v.at[slot],
                              sem_w.at[slot]).wait()

    def wait_scatter(slot):
        pltpu.make_async_copy(w_hbm.at[pl.ds(0, _CHUNK)], rows_v.at[slot],
                              sem_sc.at[slot]).wait()

    issue_idx(0, 0)
    issue_idx(1, 1)
    issue_idx(2, 2)
    wait_idx(0)
    issue_loads(0, 0)

    def step(k, _):
        slot = lax.rem(k, _NBUF)
        nslot = lax.rem(k + 1, _NBUF)

        @pl.when(k + 1 < _STEPS)
        def _():
            wait_idx(nslot)
            issue_loads(k + 1, nslot)

        wait_loads(slot)

        rs = rows_v.at[slot]
        ws = w_v.at[slot]

        @plsc.parallel_loop(0, _CHUNK, step=1, unroll=4)
        def _mul_row(e):
            for c in range(F // 16):
                sl = pl.ds(c * 16, 16)
                rs[e, sl] = rs[e, sl] * ws[e, sl]

        pltpu.async_copy(rows_v.at[slot], conv_sh.at[seg_v.at[slot]],
                         sem_sc.at[slot], add=True)

        @pl.when(k >= 1)
        def _():
            wait_scatter(lax.rem(k - 1, _NBUF))

            @pl.when(k + 2 < _STEPS)
            def _():
                issue_idx(k + 2, lax.rem(k + 2, _NBUF))

        return 0

    lax.fori_loop(0, _STEPS, step, 0)
    wait_scatter((_STEPS - 1) % _NBUF)
    plsc.subcore_barrier()

    off = sid * _ROWS_PER_TILE
    pltpu.sync_copy(conv_sh.at[pl.ds(off, _ROWS_PER_TILE)],
                    out_hbm.at[cid, pl.ds(off, _ROWS_PER_TILE)])


def _sc_conv(f, w, idx_j, seg_i, zeros):
    mesh = plsc.VectorSubcoreMesh(core_axis_name="c", subcore_axis_name="s")
    kern = functools.partial(
        pl.kernel,
        out_type=jax.ShapeDtypeStruct((_NC, _NPAD, F), jnp.float32),
        mesh=mesh,
        scratch_types=[
            pltpu.VMEM((_NBUF, _CHUNK), jnp.int32),
            pltpu.VMEM((_NBUF, _CHUNK), jnp.int32),
            pltpu.VMEM((_NBUF, _CHUNK, F), jnp.float32),
            pltpu.VMEM((_NBUF, _CHUNK, F), jnp.float32),
            pltpu.VMEM_SHARED((_NPAD, F), jnp.float32),
            pltpu.SemaphoreType.DMA((_NBUF,)),
            pltpu.SemaphoreType.DMA((_NBUF,)),
            pltpu.SemaphoreType.DMA((_NBUF,)),
            pltpu.SemaphoreType.DMA((_NBUF,)),
            pltpu.SemaphoreType.DMA((_NBUF,)),
        ],
    )(_sc_conv_body)
    return kern(f, w, idx_j, seg_i, zeros)


def _epilogue_body(p0_ref, p1_ref, x_ref, wout_ref, bout_ref, wd_ref, bd_ref,
                   y_ref, v_ref):
    conv = p0_ref[0] + p1_ref[0]
    c = _ssp(jnp.dot(conv, wout_ref[...], preferred_element_type=jnp.float32)
             + bout_ref[...])
    v = jnp.dot(c, wd_ref[...], preferred_element_type=jnp.float32) + bd_ref[...]
    v_ref[...] = v
    y_ref[...] = x_ref[...] + v


def _epilogue(parts, x, Wout, bout, Wd, bd):
    nb = N // _BN
    return pl.pallas_call(
        _epilogue_body,
        grid=(nb,),
        in_specs=[
            pl.BlockSpec((1, _BN, F), lambda i: (0, i, 0)),
            pl.BlockSpec((1, _BN, F), lambda i: (1, i, 0)),
            pl.BlockSpec((_BN, F), lambda i: (i, 0)),
            pl.BlockSpec((F, F), lambda i: (0, 0)),
            pl.BlockSpec((1, F), lambda i: (0, 0)),
            pl.BlockSpec((F, F), lambda i: (0, 0)),
            pl.BlockSpec((1, F), lambda i: (0, 0)),
        ],
        out_specs=[
            pl.BlockSpec((_BN, F), lambda i: (i, 0)),
            pl.BlockSpec((_BN, F), lambda i: (i, 0)),
        ],
        out_shape=[
            jax.ShapeDtypeStruct((N, F), jnp.float32),
            jax.ShapeDtypeStruct((N, F), jnp.float32),
        ],
    )(parts, parts, x, Wout, bout.reshape(1, F), Wd, bd.reshape(1, F))


def kernel(x, dijk, idx_j, seg_i, seg_j, seg_i_sum,
           W1, b1, W2, b2, Win, Wout, bout, Wd, bd):
    w = _filter(dijk, W1, b1, W2, b2)
    f = _in2fac(x, Win)
    zeros = jnp.zeros((_NPAD, F), jnp.float32)
    parts = _sc_conv(f, w, idx_j.astype(jnp.int32), seg_i.astype(jnp.int32),
                     zeros)
    y, v = _epilogue(parts, x, Wout, bout, Wd, bd)
    return (y, v)


# speedup vs baseline: 1.6104x; 1.0433x over previous
import functools

import jax
import jax.numpy as jnp
from jax import lax
from jax.experimental import pallas as pl
from jax.experimental.pallas import tpu as pltpu
from jax.experimental.pallas import tpu_sc as plsc

N = 10000
E = 160000
F = 128

_LOG2 = 0.6931471805599453


_LOG2E = 1.4426950408889634


def _ssp(z):
    e = jnp.exp2(jnp.abs(z) * (-_LOG2E))
    l = jnp.log2(1.0 + e)
    return jnp.maximum(z, 0.0) + (l - 1.0) * _LOG2


_BE = 1600


def _filter_body(dijk_ref, w1_ref, b1_ref, w2_ref, b2_ref, out_ref):
    h = jnp.dot(dijk_ref[...], w1_ref[...], preferred_element_type=jnp.float32)
    h = _ssp(h + b1_ref[...])
    w = jnp.dot(h, w2_ref[...], preferred_element_type=jnp.float32)
    out_ref[...] = _ssp(w + b2_ref[...])


def _filter(dijk, W1, b1, W2, b2):
    return pl.pallas_call(
        _filter_body,
        grid=(E // _BE,),
        in_specs=[
            pl.BlockSpec((_BE, F), lambda i: (i, 0)),
            pl.BlockSpec((F, F), lambda i: (0, 0)),
            pl.BlockSpec((1, F), lambda i: (0, 0)),
            pl.BlockSpec((F, F), lambda i: (0, 0)),
            pl.BlockSpec((1, F), lambda i: (0, 0)),
        ],
        out_specs=pl.BlockSpec((_BE, F), lambda i: (i, 0)),
        out_shape=jax.ShapeDtypeStruct((E, F), jnp.float32),
    )(dijk, W1, b1.reshape(1, F), W2, b2.reshape(1, F))


_BN = 1000


def _in2fac_body(x_ref, win_ref, f_ref):
    f_ref[...] = jnp.dot(x_ref[...], win_ref[...],
                         preferred_element_type=jnp.float32)


def _in2fac(x, Win):
    return pl.pallas_call(
        _in2fac_body,
        grid=(N // _BN,),
        in_specs=[
            pl.BlockSpec((_BN, F), lambda i: (i, 0)),
            pl.BlockSpec((F, F), lambda i: (0, 0)),
        ],
        out_specs=pl.BlockSpec((_BN, F), lambda i: (i, 0)),
        out_shape=jax.ShapeDtypeStruct((N, F), jnp.float32),
    )(x, Win)


_NC = 2
_NS = 16
_NW = _NC * _NS
_CHUNK = 40
_EPT = E // _NW
_STEPS = _EPT // _CHUNK
_NBUF = 3
_NPAD = 10240
_ROWS_PER_TILE = _NPAD // _NS


def _sc_conv_body(f_hbm, w_hbm, idx_hbm, seg_hbm, zeros_hbm, out_hbm,
                  idx_v, seg_v, rows_v, w_v, conv_sh,
                  sem_i, sem_s, sem_g, sem_w, sem_sc):
    cid = lax.axis_index("c")
    sid = lax.axis_index("s")
    wid = cid * _NS + sid
    ebase = wid * _EPT

    pltpu.sync_copy(zeros_hbm.at[pl.ds(sid * _ROWS_PER_TILE, _ROWS_PER_TILE)],
                    conv_sh.at[pl.ds(sid * _ROWS_PER_TILE, _ROWS_PER_TILE)])
    plsc.subcore_barrier()

    def issue_idx(k, slot):
        pltpu.async_copy(idx_hbm.at[pl.ds(ebase + k * _CHUNK, _CHUNK)],
                         idx_v.at[slot], sem_i.at[slot])
        pltpu.async_copy(seg_hbm.at[pl.ds(ebase + k * _CHUNK, _CHUNK)],
                         seg_v.at[slot], sem_s.at[slot])

    def wait_idx(slot):
        pltpu.make_async_copy(idx_hbm.at[pl.ds(0, _CHUNK)], idx_v.at[slot],
                              sem_i.at[slot]).wait()
        pltpu.make_async_copy(idx_hbm.at[pl.ds(0, _CHUNK)], seg_v.at[slot],
                              sem_s.at[slot]).wait()

    def issue_loads(k, slot):
        pltpu.async_copy(f_hbm.at[idx_v.at[slot]], rows_v.at[slot],
                         sem_g.at[slot])
        pltpu.async_copy(w_hbm.at[pl.ds(ebase + k * _CHUNK, _CHUNK)],
                         w_v.at[slot], sem_w.at[slot])

    def wait_loads(slot):
        pltpu.make_async_copy(w_hbm.at[pl.ds(0, _CHUNK)], rows_v.at[slot],
                              sem_g.at[slot]).wait()
        pltpu.make_async_copy(w_hbm.at[pl.ds(0, _CHUNK)], w_v.at[slot],
                              sem_w.at[slot]).wait()

    def wait_scatter(slot):
        pltpu.make_async_copy(w_hbm.at[pl.ds(0, _CHUNK)], rows_v.at[slot],
                              sem_sc.at[slot]).wait()

    issue_idx(0, 0)
    issue_idx(1, 1)
    issue_idx(2, 2)
    wait_idx(0)
    issue_loads(0, 0)

    def step(k, _):
        slot = lax.rem(k, _NBUF)
        nslot = lax.rem(k + 1, _NBUF)

        @pl.when(k + 1 < _STEPS)
        def _():
            wait_idx(nslot)
            issue_loads(k + 1, nslot)

        wait_loads(slot)

        rs = rows_v.at[slot]
        ws = w_v.at[slot]

        @plsc.parallel_loop(0, _CHUNK, step=1, unroll=4)
        def _mul_row(e):
            for c in range(F // 16):
                sl = pl.ds(c * 16, 16)
                rs[e, sl] = rs[e, sl] * ws[e, sl]

        pltpu.async_copy(rows_v.at[slot], conv_sh.at[seg_v.at[slot]],
                         sem_sc.at[slot], add=True)

        @pl.when(k >= 1)
        def _():
            wait_scatter(lax.rem(k - 1, _NBUF))

            @pl.when(k + 2 < _STEPS)
            def _():
                issue_idx(k + 2, lax.rem(k + 2, _NBUF))

        return 0

    lax.fori_loop(0, _STEPS, step, 0)
    wait_scatter((_STEPS - 1) % _NBUF)
    plsc.subcore_barrier()

    off = sid * _ROWS_PER_TILE
    pltpu.sync_copy(conv_sh.at[pl.ds(off, _ROWS_PER_TILE)],
                    out_hbm.at[cid, pl.ds(off, _ROWS_PER_TILE)])


def _sc_conv(f, w, idx_j, seg_i, zeros):
    mesh = plsc.VectorSubcoreMesh(core_axis_name="c", subcore_axis_name="s")
    kern = functools.partial(
        pl.kernel,
        out_type=jax.ShapeDtypeStruct((_NC, _NPAD, F), jnp.float32),
        mesh=mesh,
        scratch_types=[
            pltpu.VMEM((_NBUF, _CHUNK), jnp.int32),
            pltpu.VMEM((_NBUF, _CHUNK), jnp.int32),
            pltpu.VMEM((_NBUF, _CHUNK, F), jnp.float32),
            pltpu.VMEM((_NBUF, _CHUNK, F), jnp.float32),
            pltpu.VMEM_SHARED((_NPAD, F), jnp.float32),
            pltpu.SemaphoreType.DMA((_NBUF,)),
            pltpu.SemaphoreType.DMA((_NBUF,)),
            pltpu.SemaphoreType.DMA((_NBUF,)),
            pltpu.SemaphoreType.DMA((_NBUF,)),
            pltpu.SemaphoreType.DMA((_NBUF,)),
        ],
    )(_sc_conv_body)
    return kern(f, w, idx_j, seg_i, zeros)


def _epilogue_body(p0_ref, p1_ref, x_ref, wout_ref, bout_ref, wd_ref, bd_ref,
                   y_ref, v_ref):
    conv = p0_ref[0] + p1_ref[0]
    c = _ssp(jnp.dot(conv, wout_ref[...], preferred_element_type=jnp.float32)
             + bout_ref[...])
    v = jnp.dot(c, wd_ref[...], preferred_element_type=jnp.float32) + bd_ref[...]
    v_ref[...] = v
    y_ref[...] = x_ref[...] + v


def _epilogue(parts, x, Wout, bout, Wd, bd):
    nb = N // _BN
    return pl.pallas_call(
        _epilogue_body,
        grid=(nb,),
        in_specs=[
            pl.BlockSpec((1, _BN, F), lambda i: (0, i, 0)),
            pl.BlockSpec((1, _BN, F), lambda i: (1, i, 0)),
            pl.BlockSpec((_BN, F), lambda i: (i, 0)),
            pl.BlockSpec((F, F), lambda i: (0, 0)),
            pl.BlockSpec((1, F), lambda i: (0, 0)),
            pl.BlockSpec((F, F), lambda i: (0, 0)),
            pl.BlockSpec((1, F), lambda i: (0, 0)),
        ],
        out_specs=[
            pl.BlockSpec((_BN, F), lambda i: (i, 0)),
            pl.BlockSpec((_BN, F), lambda i: (i, 0)),
        ],
        out_shape=[
            jax.ShapeDtypeStruct((N, F), jnp.float32),
            jax.ShapeDtypeStruct((N, F), jnp.float32),
        ],
    )(parts, parts, x, Wout, bout.reshape(1, F), Wd, bd.reshape(1, F))


def kernel(x, dijk, idx_j, seg_i, seg_j, seg_i_sum,
           W1, b1, W2, b2, Win, Wout, bout, Wd, bd):
    w = _filter(dijk, W1, b1, W2, b2)
    f = _in2fac(x, Win)
    zeros = jnp.zeros((_NPAD, F), jnp.float32)
    parts = _sc_conv(f, w, idx_j.astype(jnp.int32), seg_i.astype(jnp.int32),
                     zeros)
    y, v = _epilogue(parts, x, Wout, bout, Wd, bd)
    return (y, v)
